# baseline (speedup 1.0000x reference)
import functools

import jax
import jax.numpy as jnp
from jax import lax
from jax.experimental import pallas as pl
from jax.experimental.pallas import tpu as pltpu
from jax.experimental.pallas import tpu_sc as plsc

N_NODES = 10000
N_EDGES = 320000
D_ATOM = 128
D_BOND = 16
H = 128

NUM_CORES = 2
NUM_SUBCORES = 16
NW = NUM_CORES * NUM_SUBCORES
COLS = D_BOND // NUM_CORES
NCHK = N_EDGES // 128
CPS = NCHK // NUM_SUBCORES
BCHK = 13
NBLOCKS = CPS // BCHK
N_PAD = 10112
ROWS = N_PAD // 128
FLAT = N_PAD * COLS // 128


def _sc_segment_sum(dst, attr_t, zeros_acc):
    mesh = plsc.VectorSubcoreMesh(
        core_axis_name="c", subcore_axis_name="s",
        num_cores=NUM_CORES, num_subcores=NUM_SUBCORES)

    @functools.partial(
        pl.kernel,
        out_type=jax.ShapeDtypeStruct((NW, COLS, ROWS, 128), jnp.float32),
        mesh=mesh,
        compiler_params=pltpu.CompilerParams(
            use_tc_tiling_on_sc=False, needs_layout_passes=False),
        scratch_types=[
            pltpu.VMEM((BCHK * 128,), jnp.int32),
            pltpu.VMEM((BCHK * 128,), jnp.int32),
            pltpu.VMEM((COLS, BCHK, 128), jnp.float32),
            pltpu.VMEM((COLS, BCHK, 128), jnp.float32),
            pltpu.VMEM((COLS, ROWS, 128), jnp.float32),
            pltpu.SemaphoreType.DMA,
            pltpu.SemaphoreType.DMA,
        ],
    )
    def sc_kernel(dst_hbm, attr_hbm, zero_hbm, out_hbm, idx0_v, idx1_v,
                  attr0_v, attr1_v, acc_v, isem, asem):
        cid = lax.axis_index("c")
        sid = lax.axis_index("s")
        wid = cid * NUM_SUBCORES + sid
        j0 = cid * COLS
        lane = lax.iota(jnp.int32, 16)

        pltpu.sync_copy(zero_hbm, acc_v)
        idx_bufs = (idx0_v, idx1_v)
        attr_bufs = (attr0_v, attr1_v)

        def start_loads(b):
            c0 = sid * CPS + b * BCHK
            di = pltpu.async_copy(dst_hbm.at[pl.ds(c0 * 128, BCHK * 128)],
                                  idx_bufs[b % 2], isem)
            da = pltpu.async_copy(
                attr_hbm.at[pl.ds(j0, COLS), pl.ds(c0, BCHK), :],
                attr_bufs[b % 2], asem)
            return di, da

        def do_groups(idx_v, attr_v, c, nvec=8):
            for u in range(nvec):
                rv = idx_v[pl.ds(c * 128 + u * 16, 16)]
                rhi = lax.shift_right_logical(rv, 7)
                rlo = lax.bitwise_and(rv, 127)
                for j in range(COLS):
                    vals = attr_v[j, c, pl.ds(u * 16, 16)]
                    colj = jnp.full((16,), j, jnp.int32)
                    plsc.addupdate_scatter(acc_v, [colj, rhi, rlo], vals)

        pending = start_loads(0)
        for b in range(NBLOCKS):
            idx_v = idx_bufs[b % 2]
            attr_v = attr_bufs[b % 2]
            pending[0].wait()
            pending[1].wait()
            if b + 1 < NBLOCKS:
                pending = start_loads(b + 1)

            def chunk_body(c, carry, idx_v=idx_v, attr_v=attr_v):
                do_groups(idx_v, attr_v, c)
                return carry

            lax.fori_loop(0, BCHK, chunk_body, 0)

        @pl.when(sid < 4)
        def _():
            ec = NCHK - 4 + sid
            pltpu.sync_copy(dst_hbm.at[pl.ds(ec * 128, 128)],
                            idx0_v.at[pl.ds(0, 128)])
            pltpu.sync_copy(attr_hbm.at[pl.ds(j0, COLS), pl.ds(ec, 1), :],
                            attr0_v.at[:, pl.ds(0, 1), :])
            do_groups(idx0_v, attr0_v, 0)

        pltpu.sync_copy(acc_v, out_hbm.at[wid])

    return sc_kernel(dst, attr_t, zeros_acc)


def _tc_reduce(partials):
    def red_kernel(p_ref, out_ref):
        out_ref[0] = jnp.sum(p_ref[:NUM_SUBCORES], axis=0)
        out_ref[1] = jnp.sum(p_ref[NUM_SUBCORES:], axis=0)

    return pl.pallas_call(
        red_kernel,
        out_shape=jax.ShapeDtypeStruct((NUM_CORES, COLS, ROWS, 128),
                                       jnp.float32),
    )(partials)


def _tc_finish(x, agg0, agg1, WaT, WbT0, WbT1, ba, gamma, beta):
    def tc_kernel(x_ref, a0_ref, a1_ref, wat_ref, wbt0_ref, wbt1_ref,
                  ba_ref, g_ref, b_ref, out_ref):
        h = jnp.dot(x_ref[...], wat_ref[...],
                    preferred_element_type=jnp.float32)
        h = h + jnp.dot(a0_ref[...], wbt0_ref[...],
                        preferred_element_type=jnp.float32)
        h = h + jnp.dot(a1_ref[...], wbt1_ref[...],
                        preferred_element_type=jnp.float32)
        h = jnp.maximum(h + ba_ref[...], 0.0)
        mean = jnp.mean(h, axis=0, keepdims=True)
        var = jnp.mean(h * h, axis=0, keepdims=True) - mean * mean
        inv = lax.rsqrt(var + 1e-5)
        out_ref[...] = (h - mean) * (inv * g_ref[...]) + b_ref[...]

    return pl.pallas_call(
        tc_kernel,
        grid=(1,),
        in_specs=[
            pl.BlockSpec((N_NODES, D_ATOM), lambda i: (0, 0)),
            pl.BlockSpec((N_NODES, COLS), lambda i: (0, 0)),
            pl.BlockSpec((N_NODES, COLS), lambda i: (0, 0)),
            pl.BlockSpec((D_ATOM, H), lambda i: (0, 0)),
            pl.BlockSpec((COLS, H), lambda i: (0, 0)),
            pl.BlockSpec((COLS, H), lambda i: (0, 0)),
            pl.BlockSpec((1, H), lambda i: (0, 0)),
            pl.BlockSpec((1, H), lambda i: (0, 0)),
            pl.BlockSpec((1, H), lambda i: (0, 0)),
        ],
        out_specs=pl.BlockSpec((N_NODES, H), lambda i: (0, 0)),
        out_shape=jax.ShapeDtypeStruct((N_NODES, H), jnp.float32),
    )(x, agg0, agg1, WaT, WbT0, WbT1, ba, gamma, beta)


def kernel(x, edge_index, edge_attr, Wa, ba, Wb, bb, gamma, beta):
    dst = edge_index[1].astype(jnp.int32)
    attr_t = edge_attr.T.reshape(D_BOND, NCHK, 128)
    zeros_acc = jnp.zeros((COLS, ROWS, 128), jnp.float32)
    partials = _sc_segment_sum(dst, attr_t, zeros_acc)
    red = _tc_reduce(partials)
    red = red.reshape(NUM_CORES, COLS, N_PAD)
    agg0 = red[0].T
    agg1 = red[1].T
    WbT = Wb.T
    return _tc_finish(x, agg0, agg1, Wa.T, WbT[:COLS], WbT[COLS:],
                      ba.reshape(1, H), gamma.reshape(1, H),
                      beta.reshape(1, H))


# speedup vs baseline: 1.0001x; 1.0001x over previous
import functools

import jax
import jax.numpy as jnp
from jax import lax
from jax.experimental import pallas as pl
from jax.experimental.pallas import tpu as pltpu
from jax.experimental.pallas import tpu_sc as plsc

N_NODES = 10000
N_EDGES = 320000
D_ATOM = 128
D_BOND = 16
H = 128

NUM_CORES = 2
NUM_SUBCORES = 16
NW = NUM_CORES * NUM_SUBCORES
COLS = D_BOND // NUM_CORES
EDGES_PER_SLICE = N_EDGES // NUM_SUBCORES
BLOCK = 800
NBLOCKS = EDGES_PER_SLICE // BLOCK
GROUPS = BLOCK // 16
UNROLL = 5
N_PAD = 10112
FLAT = N_PAD * COLS // 128


def _sc_segment_sum(dst, attr_t, zeros_acc):
    mesh = plsc.VectorSubcoreMesh(
        core_axis_name="c", subcore_axis_name="s",
        num_cores=NUM_CORES, num_subcores=NUM_SUBCORES)

    @functools.partial(
        pl.kernel,
        out_type=jax.ShapeDtypeStruct((NW, COLS, N_PAD), jnp.float32),
        mesh=mesh,
        compiler_params=pltpu.CompilerParams(
            use_tc_tiling_on_sc=False, needs_layout_passes=False),
        scratch_types=[
            pltpu.VMEM((BLOCK,), jnp.int32),
            pltpu.VMEM((BLOCK,), jnp.int32),
            pltpu.VMEM((COLS, BLOCK), jnp.float32),
            pltpu.VMEM((COLS, BLOCK), jnp.float32),
            pltpu.VMEM((COLS, N_PAD), jnp.float32),
            pltpu.SemaphoreType.DMA,
            pltpu.SemaphoreType.DMA,
        ],
    )
    def sc_kernel(dst_hbm, attr_hbm, zero_hbm, out_hbm, idx0_v, idx1_v,
                  attr0_v, attr1_v, acc_v, isem, asem):
        cid = lax.axis_index("c")
        sid = lax.axis_index("s")
        wid = cid * NUM_SUBCORES + sid
        j0 = cid * COLS
        ebase = sid * EDGES_PER_SLICE
        lane = lax.iota(jnp.int32, 16)

        pltpu.sync_copy(zero_hbm, acc_v)
        idx_bufs = (idx0_v, idx1_v)
        attr_bufs = (attr0_v, attr1_v)

        def start_loads(b):
            off = ebase + b * BLOCK
            di = pltpu.async_copy(dst_hbm.at[pl.ds(off, BLOCK)],
                                  idx_bufs[b % 2], isem)
            da = pltpu.async_copy(
                attr_hbm.at[pl.ds(j0, COLS), pl.ds(off, BLOCK)],
                attr_bufs[b % 2], asem)
            return di, da

        pending = start_loads(0)
        for b in range(NBLOCKS):
            idx_v = idx_bufs[b % 2]
            attr_v = attr_bufs[b % 2]
            pending[0].wait()
            pending[1].wait()
            if b + 1 < NBLOCKS:
                pending = start_loads(b + 1)

            def group_body(g, carry, idx_v=idx_v, attr_v=attr_v):
                for u in range(UNROLL):
                    gb = (g * UNROLL + u) * 16
                    rv = idx_v[pl.ds(gb, 16)]
                    for j in range(COLS):
                        vals = attr_v[j, pl.ds(gb, 16)]
                        dstcol = jnp.full((16,), j, jnp.int32)
                        plsc.addupdate_scatter(acc_v, [dstcol, rv], vals)
                return carry

            lax.fori_loop(0, GROUPS // UNROLL, group_body, 0)
        pltpu.sync_copy(acc_v, out_hbm.at[wid])

    return sc_kernel(dst, attr_t, zeros_acc)


def _tc_reduce(partials_flat):
    def red_kernel(p_ref, out_ref):
        out_ref[0] = jnp.sum(p_ref[:NUM_SUBCORES], axis=0)
        out_ref[1] = jnp.sum(p_ref[NUM_SUBCORES:], axis=0)

    return pl.pallas_call(
        red_kernel,
        out_shape=jax.ShapeDtypeStruct((NUM_CORES, FLAT, 128), jnp.float32),
    )(partials_flat)


def _tc_node_linear(x, WaT, ba):
    def nl_kernel(x_ref, wat_ref, ba_ref, out_ref):
        out_ref[...] = jnp.dot(x_ref[...], wat_ref[...],
                               preferred_element_type=jnp.float32) + ba_ref[...]

    return pl.pallas_call(
        nl_kernel,
        out_shape=jax.ShapeDtypeStruct((N_NODES, H), jnp.float32),
    )(x, WaT, ba)


def _tc_finish(hx, agg0, agg1, WbT0, WbT1, gamma, beta):
    def tc_kernel(hx_ref, a0_ref, a1_ref, wbt0_ref, wbt1_ref,
                  g_ref, b_ref, out_ref):
        h = hx_ref[...]
        h = h + jnp.dot(a0_ref[...], wbt0_ref[...],
                        preferred_element_type=jnp.float32)
        h = h + jnp.dot(a1_ref[...], wbt1_ref[...],
                        preferred_element_type=jnp.float32)
        h = jnp.maximum(h, 0.0)
        mean = jnp.mean(h, axis=0, keepdims=True)
        var = jnp.mean(h * h, axis=0, keepdims=True) - mean * mean
        inv = lax.rsqrt(var + 1e-5)
        out_ref[...] = (h - mean) * (inv * g_ref[...]) + b_ref[...]

    return pl.pallas_call(
        tc_kernel,
        grid=(1,),
        in_specs=[
            pl.BlockSpec((N_NODES, H), lambda i: (0, 0)),
            pl.BlockSpec((N_NODES, COLS), lambda i: (0, 0)),
            pl.BlockSpec((N_NODES, COLS), lambda i: (0, 0)),
            pl.BlockSpec((COLS, H), lambda i: (0, 0)),
            pl.BlockSpec((COLS, H), lambda i: (0, 0)),
            pl.BlockSpec((1, H), lambda i: (0, 0)),
            pl.BlockSpec((1, H), lambda i: (0, 0)),
        ],
        out_specs=pl.BlockSpec((N_NODES, H), lambda i: (0, 0)),
        out_shape=jax.ShapeDtypeStruct((N_NODES, H), jnp.float32),
    )(hx, agg0, agg1, WbT0, WbT1, gamma, beta)


def kernel(x, edge_index, edge_attr, Wa, ba, Wb, bb, gamma, beta):
    dst = edge_index[1].astype(jnp.int32)
    attr_t = edge_attr.T
    zeros_acc = jnp.zeros((COLS, N_PAD), jnp.float32)
    partials = _sc_segment_sum(dst, attr_t, zeros_acc)
    hx = _tc_node_linear(x, Wa.T, ba.reshape(1, H))
    red = _tc_reduce(partials.reshape(NW, FLAT, 128))
    red = red.reshape(NUM_CORES, COLS, N_PAD)
    agg0 = red[0].T
    agg1 = red[1].T
    WbT = Wb.T
    return _tc_finish(hx, agg0, agg1, WbT[:COLS], WbT[COLS:],
                      gamma.reshape(1, H), beta.reshape(1, H))


# speedup vs baseline: 1.0134x; 1.0133x over previous
import functools

import jax
import jax.numpy as jnp
from jax import lax
from jax.experimental import pallas as pl
from jax.experimental.pallas import tpu as pltpu
from jax.experimental.pallas import tpu_sc as plsc

N_NODES = 10000
N_EDGES = 320000
D_ATOM = 128
D_BOND = 16
H = 128

NUM_CORES = 2
NUM_SUBCORES = 16
NW = NUM_CORES * NUM_SUBCORES
COLS = D_BOND // NUM_CORES
EDGES_PER_SLICE = N_EDGES // NUM_SUBCORES
BLOCK = 2000
NBLOCKS = EDGES_PER_SLICE // BLOCK
GROUPS = BLOCK // 16
UNROLL = 5
N_PAD = 10112
FLAT = N_PAD * COLS // 128


def _sc_segment_sum(dst, attr_t, zeros_acc):
    mesh = plsc.VectorSubcoreMesh(
        core_axis_name="c", subcore_axis_name="s",
        num_cores=NUM_CORES, num_subcores=NUM_SUBCORES)

    @functools.partial(
        pl.kernel,
        out_type=jax.ShapeDtypeStruct((NW, COLS, N_PAD), jnp.float32),
        mesh=mesh,
        compiler_params=pltpu.CompilerParams(
            use_tc_tiling_on_sc=False, needs_layout_passes=False),
        scratch_types=[
            pltpu.VMEM((BLOCK,), jnp.int32),
            pltpu.VMEM((BLOCK,), jnp.int32),
            pltpu.VMEM((COLS, BLOCK), jnp.float32),
            pltpu.VMEM((COLS, BLOCK), jnp.float32),
            pltpu.VMEM((COLS, N_PAD), jnp.float32),
            pltpu.SemaphoreType.DMA,
            pltpu.SemaphoreType.DMA,
        ],
    )
    def sc_kernel(dst_hbm, attr_hbm, zero_hbm, out_hbm, idx0_v, idx1_v,
                  attr0_v, attr1_v, acc_v, isem, asem):
        cid = lax.axis_index("c")
        sid = lax.axis_index("s")
        wid = cid * NUM_SUBCORES + sid
        j0 = cid * COLS
        ebase = sid * EDGES_PER_SLICE
        lane = lax.iota(jnp.int32, 16)

        pltpu.sync_copy(zero_hbm, acc_v)
        idx_bufs = (idx0_v, idx1_v)
        attr_bufs = (attr0_v, attr1_v)

        def start_loads(b):
            off = ebase + b * BLOCK
            di = pltpu.async_copy(dst_hbm.at[pl.ds(off, BLOCK)],
                                  idx_bufs[b % 2], isem)
            da = pltpu.async_copy(
                attr_hbm.at[pl.ds(j0, COLS), pl.ds(off, BLOCK)],
                attr_bufs[b % 2], asem)
            return di, da

        pending = start_loads(0)
        for b in range(NBLOCKS):
            idx_v = idx_bufs[b % 2]
            attr_v = attr_bufs[b % 2]
            pending[0].wait()
            pending[1].wait()
            if b + 1 < NBLOCKS:
                pending = start_loads(b + 1)

            def group_body(g, carry, idx_v=idx_v, attr_v=attr_v):
                for u in range(UNROLL):
                    gb = (g * UNROLL + u) * 16
                    rv = idx_v[pl.ds(gb, 16)]
                    for j in range(COLS):
                        vals = attr_v[j, pl.ds(gb, 16)]
                        dstcol = jnp.full((16,), j, jnp.int32)
                        plsc.addupdate_scatter(acc_v, [dstcol, rv], vals)
                return carry

            lax.fori_loop(0, GROUPS // UNROLL, group_body, 0)
        pltpu.sync_copy(acc_v, out_hbm.at[wid])

    return sc_kernel(dst, attr_t, zeros_acc)


CONTRACT0 = (((0,), (0,)), ((), ()))


def _tc_all(x, partials, WaT, WbT0, WbT1, ba, gamma, beta):
    def tc_kernel(x_ref, p_ref, wat_ref, wbt0_ref, wbt1_ref,
                  ba_ref, g_ref, b_ref, out_ref):
        red0 = jnp.sum(p_ref[:NUM_SUBCORES], axis=0)
        red1 = jnp.sum(p_ref[NUM_SUBCORES:], axis=0)
        h = jnp.dot(x_ref[...], wat_ref[...],
                    preferred_element_type=jnp.float32)
        h2 = lax.dot_general(red0, wbt0_ref[...], CONTRACT0,
                             preferred_element_type=jnp.float32)
        h3 = lax.dot_general(red1, wbt1_ref[...], CONTRACT0,
                             preferred_element_type=jnp.float32)
        h = h + h2[:N_NODES] + h3[:N_NODES]
        h = jnp.maximum(h + ba_ref[...], 0.0)
        mean = jnp.mean(h, axis=0, keepdims=True)
        var = jnp.mean(h * h, axis=0, keepdims=True) - mean * mean
        inv = lax.rsqrt(var + 1e-5)
        out_ref[...] = (h - mean) * (inv * g_ref[...]) + b_ref[...]

    return pl.pallas_call(
        tc_kernel,
        grid=(1,),
        in_specs=[
            pl.BlockSpec((N_NODES, D_ATOM), lambda i: (0, 0)),
            pl.BlockSpec((NW, COLS, N_PAD), lambda i: (0, 0, 0)),
            pl.BlockSpec((D_ATOM, H), lambda i: (0, 0)),
            pl.BlockSpec((COLS, H), lambda i: (0, 0)),
            pl.BlockSpec((COLS, H), lambda i: (0, 0)),
            pl.BlockSpec((1, H), lambda i: (0, 0)),
            pl.BlockSpec((1, H), lambda i: (0, 0)),
            pl.BlockSpec((1, H), lambda i: (0, 0)),
        ],
        out_specs=pl.BlockSpec((N_NODES, H), lambda i: (0, 0)),
        out_shape=jax.ShapeDtypeStruct((N_NODES, H), jnp.float32),
    )(x, partials, WaT, WbT0, WbT1, ba, gamma, beta)


def kernel(x, edge_index, edge_attr, Wa, ba, Wb, bb, gamma, beta):
    dst = edge_index[1].astype(jnp.int32)
    attr_t = edge_attr.T
    zeros_acc = jnp.zeros((COLS, N_PAD), jnp.float32)
    partials = _sc_segment_sum(dst, attr_t, zeros_acc)
    WbT = Wb.T
    return _tc_all(x, partials, Wa.T, WbT[:COLS], WbT[COLS:],
                   ba.reshape(1, H), gamma.reshape(1, H),
                   beta.reshape(1, H))
